# baseline (speedup 1.0000x reference)
import jax
import jax.numpy as jnp
from jax.experimental import pallas as pl
from jax.experimental.pallas import tpu as pltpu

_B, _J, _H, _W = 128, 17, 64, 48
_HW = _H * _W
_TOPK = 8
_BB = 16


def _ohkm_kernel(o_ref, t_ref, w_ref, out_ref):
    d = o_ref[...] - t_ref[...]
    s = jnp.sum(d * d, axis=2)
    w = w_ref[...]
    vals = s * (w * w) * (0.5 / _HW)
    col = jax.lax.broadcasted_iota(jnp.int32, vals.shape, 1)
    acc = jnp.zeros((vals.shape[0],), jnp.float32)
    neg_inf = jnp.float32(-jnp.inf)
    for _ in range(_TOPK):
        m = jnp.max(vals, axis=1)
        acc = acc + m
        is_max = vals == m[:, None]
        first_idx = jnp.min(jnp.where(is_max, col, _J), axis=1)
        vals = jnp.where(col == first_idx[:, None], neg_inf, vals)
    partial = jnp.sum(acc) * (1.0 / (_TOPK * _B))

    @pl.when(pl.program_id(0) == 0)
    def _():
        out_ref[0, 0] = 0.0

    out_ref[0, 0] += partial


def kernel(outs, targets, target_weights):
    o = outs.reshape(_B, _J, _HW)
    t = targets.reshape(_B, _J, _HW)
    w = target_weights.reshape(_B, _J)
    out = pl.pallas_call(
        _ohkm_kernel,
        grid=(_B // _BB,),
        in_specs=[
            pl.BlockSpec((_BB, _J, _HW), lambda i: (i, 0, 0)),
            pl.BlockSpec((_BB, _J, _HW), lambda i: (i, 0, 0)),
            pl.BlockSpec((_BB, _J), lambda i: (i, 0)),
        ],
        out_specs=pl.BlockSpec(
            (1, 1), lambda i: (0, 0), memory_space=pltpu.SMEM
        ),
        out_shape=jax.ShapeDtypeStruct((1, 1), jnp.float32),
    )(o, t, w)
    return out.reshape(())
